# baseline (speedup 1.0000x reference)
import jax
import jax.numpy as jnp
from jax.experimental import pallas as pl

N = 16384
NUM_CLASSES = 1000
ROWS_PER_BLOCK = 512


def _onehot_block(x_ref, o_ref):
    cols = jax.lax.broadcasted_iota(jnp.int32, (ROWS_PER_BLOCK, NUM_CLASSES), 1)
    o_ref[...] = (x_ref[...] == cols).astype(jnp.int32)


def kernel(x):
    x2 = x.reshape(N, 1).astype(jnp.int32)
    grid = N // ROWS_PER_BLOCK
    return pl.pallas_call(
        _onehot_block,
        grid=(grid,),
        in_specs=[pl.BlockSpec((ROWS_PER_BLOCK, 1), lambda i: (i, 0))],
        out_specs=pl.BlockSpec((ROWS_PER_BLOCK, NUM_CLASSES), lambda i: (i, 0)),
        out_shape=jax.ShapeDtypeStruct((N, NUM_CLASSES), jnp.int32),
    )(x2)
